# baseline (speedup 1.0000x reference)
import functools

import jax
import jax.numpy as jnp
from jax import lax
from jax.experimental import pallas as pl
from jax.experimental.pallas import tpu as pltpu
from jax.experimental.pallas import tpu_sc as plsc

N = 10000
E = 320000
D = 128
F1, F2, F3 = 64, 32, 16
T = 16
NP = 10240

_INFO = plsc.get_sparse_core_info()
NC = _INFO.num_cores
NS = _INFO.num_subcores
NW = NC * NS
EW = E // NW
C = 125
NCH = EW // C
K = 4
NT = NCH // K
RPT = NP // NS

_mesh = plsc.VectorSubcoreMesh(core_axis_name="c", subcore_axis_name="s")


def _sc_deg(dst1_hbm, dst2_hbm, ones_hbm, zrow_hbm, degp1_hbm, degp2_hbm,
            dsta, onesv, acc1, acc2, sem):
    c = lax.axis_index("c")
    s = lax.axis_index("s")
    wid = s * NC + c

    pltpu.sync_copy(zrow_hbm, acc1.at[pl.ds(s * RPT, RPT)])
    pltpu.sync_copy(zrow_hbm, acc2.at[pl.ds(s * RPT, RPT)])
    pltpu.sync_copy(ones_hbm, onesv)
    plsc.subcore_barrier()

    for gi, (dh, acc) in enumerate(((dst1_hbm, acc1), (dst2_hbm, acc2))):
        pltpu.sync_copy(dh.at[pl.ds(wid * NCH, NCH)], dsta.at[gi])
        for grp in range(0, NCH, 20):
            descs = [pltpu.async_copy(onesv, acc.at[dsta.at[gi, j]], sem,
                                      add=True)
                     for j in range(grp, grp + 20)]
            for d in descs:
                d.wait()

    plsc.subcore_barrier()
    pltpu.sync_copy(acc1.at[pl.ds(s * RPT, RPT)],
                    degp1_hbm.at[pl.ds(c * NP + s * RPT, RPT)])
    pltpu.sync_copy(acc2.at[pl.ds(s * RPT, RPT)],
                    degp2_hbm.at[pl.ds(c * NP + s * RPT, RPT)])


@functools.partial(
    pl.kernel,
    out_type=(jax.ShapeDtypeStruct((2 * NP, 16), jnp.float32),
              jax.ShapeDtypeStruct((2 * NP, 16), jnp.float32)),
    mesh=_mesh,
    compiler_params=pltpu.CompilerParams(use_tc_tiling_on_sc=False),
    scratch_types=[
        pltpu.VMEM((2, NCH, C), jnp.int32),
        pltpu.VMEM((C, 16), jnp.float32),
        pltpu.VMEM_SHARED((NP, 16), jnp.float32),
        pltpu.VMEM_SHARED((NP, 16), jnp.float32),
        pltpu.SemaphoreType.DMA,
    ],
)
def _deg_call(dst1_hbm, dst2_hbm, ones_hbm, zrow_hbm, degp1_hbm, degp2_hbm,
              dsta, onesv, acc1, acc2, sem):
    _sc_deg(dst1_hbm, dst2_hbm, ones_hbm, zrow_hbm, degp1_hbm, degp2_hbm,
            dsta, onesv, acc1, acc2, sem)


GS = 624


def _sc_scatter_body(src1_hbm, dst1_hbm, src2_hbm, dst2_hbm, g1_hbm, g2_hbm,
                     zrow_hbm, out1_hbm, out2_hbm,
                     srca, dsta, rows, g_sp, acc, gsems, ssems, Kf, IH):
    c = lax.axis_index("c")
    s = lax.axis_index("s")
    wid = s * NC + c
    NCHh = NCH // IH
    NTh = NCHh // Kf

    for (sh, dh, gh, oh) in ((src1_hbm, dst1_hbm, g1_hbm, out1_hbm),
                             (src2_hbm, dst2_hbm, g2_hbm, out2_hbm)):
        pltpu.sync_copy(zrow_hbm, acc.at[pl.ds(s * RPT, RPT)])
        pltpu.sync_copy(gh.at[pl.ds(s * GS, GS)], g_sp.at[pl.ds(s * GS, GS)])

        @pl.when(s == NS - 1)
        def _():
            pltpu.sync_copy(gh.at[pl.ds(NS * GS, N - NS * GS)],
                            g_sp.at[pl.ds(NS * GS, N - NS * GS)])

        plsc.subcore_barrier()

        for h in range(IH):
            pltpu.sync_copy(sh.at[pl.ds(wid * NCH + h * NCHh, NCHh)], srca)
            pltpu.sync_copy(dh.at[pl.ds(wid * NCH + h * NCHh, NCHh)], dsta)

            def fire(t):
                p = t & 1
                return [pltpu.async_copy(g_sp.at[srca.at[t * Kf + b]],
                                         rows.at[p, b], gsems[p])
                        for b in range(Kf)]

            gd = {0: fire(0), 1: fire(1)}
            for t in range(NTh):
                p = t & 1
                for d in gd.pop(t):
                    d.wait()
                sd = [pltpu.async_copy(rows.at[p, b],
                                       acc.at[dsta.at[t * Kf + b]],
                                       ssems[p], add=True)
                      for b in range(Kf)]
                for d in sd:
                    d.wait()
                if t + 2 < NTh:
                    gd[t + 2] = fire(t + 2)

        plsc.subcore_barrier()
        pltpu.sync_copy(acc.at[pl.ds(s * RPT, RPT)],
                        oh.at[pl.ds(c * NP + s * RPT, RPT)])


def _make_scatter(F):
    Kf = 2 if F == F1 else 4
    IH = 2 if F == F1 else 1

    @functools.partial(
        pl.kernel,
        out_type=(jax.ShapeDtypeStruct((2 * NP, F), jnp.float32),
                  jax.ShapeDtypeStruct((2 * NP, F), jnp.float32)),
        mesh=_mesh,
        compiler_params=pltpu.CompilerParams(use_tc_tiling_on_sc=False),
        scratch_types=[
            pltpu.VMEM((NCH // IH, C), jnp.int32),
            pltpu.VMEM((NCH // IH, C), jnp.int32),
            pltpu.VMEM((2, Kf, C, F), jnp.float32),
            pltpu.VMEM_SHARED((N, F), jnp.float32),
            pltpu.VMEM_SHARED((NP, F), jnp.float32),
            (pltpu.SemaphoreType.DMA, pltpu.SemaphoreType.DMA),
            (pltpu.SemaphoreType.DMA, pltpu.SemaphoreType.DMA),
        ],
    )
    def call(src1_hbm, dst1_hbm, src2_hbm, dst2_hbm, g1_hbm, g2_hbm,
             zrow_hbm, out1_hbm, out2_hbm,
             srca, dsta, rows, g_sp, acc, gsems, ssems):
        _sc_scatter_body(src1_hbm, dst1_hbm, src2_hbm, dst2_hbm, g1_hbm,
                         g2_hbm, zrow_hbm, out1_hbm, out2_hbm,
                         srca, dsta, rows, g_sp, acc, gsems, ssems, Kf, IH)

    return call


_scatter = {F: _make_scatter(F) for F in (F1, F2, F3)}


def _dinv_of(degp_ref):
    deg = degp_ref[pl.ds(0, N)] + degp_ref[pl.ds(NP, N)] + 1.0
    return lax.rsqrt(jnp.maximum(deg, 1.0))


def _tc_prep_body(x1_ref, x2_ref, w_ref, degp1_ref, degp2_ref, g1_ref, g2_ref):
    w = w_ref[...]
    g1_ref[...] = jnp.dot(x1_ref[...], w,
                          preferred_element_type=jnp.float32) * _dinv_of(degp1_ref)
    g2_ref[...] = jnp.dot(x2_ref[...], w,
                          preferred_element_type=jnp.float32) * _dinv_of(degp2_ref)


def _tc_comb_body(p1_ref, p2_ref, g1_ref, g2_ref, degp1_ref, degp2_ref,
                  b_ref, w_ref, o1_ref, o2_ref):
    w = w_ref[...]
    b = b_ref[...]
    for (p_ref, g_ref, degp_ref, o_ref) in (
            (p1_ref, g1_ref, degp1_ref, o1_ref),
            (p2_ref, g2_ref, degp2_ref, o2_ref)):
        dinv = _dinv_of(degp_ref)
        srow = p_ref[pl.ds(0, N)] + p_ref[pl.ds(NP, N)] + g_ref[...]
        x = jnp.maximum(srow * dinv + b, 0.0)
        o_ref[...] = jnp.dot(x, w, preferred_element_type=jnp.float32) * dinv


def _tc_last_head_body(p1_ref, p2_ref, g1_ref, g2_ref, degp1_ref, degp2_ref,
                       b_ref, watt_ref, wtT_ref, vtT_ref, bt_ref,
                       wfc_ref, bfc_ref, wsc_ref, bsc_ref, o_ref):
    b = b_ref[...]
    watt = watt_ref[...]

    def pool(p_ref, g_ref, degp_ref):
        dinv = _dinv_of(degp_ref)
        h = (p_ref[pl.ds(0, N)] + p_ref[pl.ds(NP, N)] + g_ref[...]) * dinv + b
        mean = jnp.sum(h, axis=0, keepdims=True) * (1.0 / N)
        ctx = jnp.tanh(jnp.dot(mean, watt, preferred_element_type=jnp.float32))
        sc = jax.nn.sigmoid(jnp.sum(h * ctx, axis=1, keepdims=True))
        return jnp.sum(h * sc, axis=0, keepdims=True)

    e1 = pool(p1_ref, g1_ref, degp1_ref)
    e2 = pool(p2_ref, g2_ref, degp2_ref)
    parts = []
    for k in range(T):
        a = jnp.dot(e1, wtT_ref[k], preferred_element_type=jnp.float32)
        parts.append(jnp.sum(a * e2, axis=1, keepdims=True))
    scoring = jnp.concatenate(parts, axis=1)
    e12 = jnp.concatenate([e1, e2], axis=1)
    block = jnp.dot(e12, vtT_ref[...], preferred_element_type=jnp.float32) \
        + bt_ref[...]
    combined = jnp.maximum(scoring + block, 0.0)
    feat = jnp.maximum(
        jnp.dot(combined, wfc_ref[...], preferred_element_type=jnp.float32)
        + bfc_ref[...], 0.0)
    o_ref[...] = jax.nn.sigmoid(
        jnp.dot(feat, wsc_ref[...], preferred_element_type=jnp.float32)
        + bsc_ref[...])


_TC_PARAMS = pltpu.CompilerParams(vmem_limit_bytes=100 * 1024 * 1024)


def _tc_prep(x1, x2, w, degp1, degp2):
    return pl.pallas_call(
        _tc_prep_body,
        out_shape=(jax.ShapeDtypeStruct((N, w.shape[1]), jnp.float32),
                   jax.ShapeDtypeStruct((N, w.shape[1]), jnp.float32)),
        compiler_params=_TC_PARAMS,
    )(x1, x2, w, degp1, degp2)


def _tc_comb(p1, p2, g1, g2, degp1, degp2, b, w):
    return pl.pallas_call(
        _tc_comb_body,
        out_shape=(jax.ShapeDtypeStruct((N, w.shape[1]), jnp.float32),
                   jax.ShapeDtypeStruct((N, w.shape[1]), jnp.float32)),
        compiler_params=_TC_PARAMS,
    )(p1, p2, g1, g2, degp1, degp2, b, w)


def _tc_last_head(p1, p2, g1, g2, degp1, degp2, b, watt, wtT, vtT, bt,
                  wfc, bfc, wsc, bsc):
    return pl.pallas_call(
        _tc_last_head_body,
        out_shape=jax.ShapeDtypeStruct((1, 1), jnp.float32),
        compiler_params=_TC_PARAMS,
    )(p1, p2, g1, g2, degp1, degp2, b, watt, wtT, vtT, bt, wfc, bfc, wsc, bsc)


def kernel(features_1, edge_index_1, batch_1, features_2, edge_index_2,
           batch_2, W1, b1, W2, b2, W3, b3, Watt, Wt, Vt, bt, Wfc, bfc,
           Wsc, bsc):
    src1 = edge_index_1[0].reshape(E // C, C)
    dst1 = edge_index_1[1].reshape(E // C, C)
    src2 = edge_index_2[0].reshape(E // C, C)
    dst2 = edge_index_2[1].reshape(E // C, C)

    ones16 = jnp.ones((C, 16), jnp.float32)
    zrow = {F: jnp.zeros((RPT, F), jnp.float32) for F in (16, F1, F2, F3)}

    degp1, degp2 = _deg_call(dst1, dst2, ones16, zrow[16])
    degp1 = degp1[:, :1]
    degp2 = degp2[:, :1]

    g1, g2 = _tc_prep(features_1, features_2, W1, degp1, degp2)
    for b, wn, F in ((b1.reshape(1, F1), W2, F1), (b2.reshape(1, F2), W3, F2)):
        p1, p2 = _scatter[F](src1, dst1, src2, dst2, g1, g2, zrow[F])
        g1, g2 = _tc_comb(p1, p2, g1, g2, degp1, degp2, b, wn)
    p1, p2 = _scatter[F3](src1, dst1, src2, dst2, g1, g2, zrow[F3])

    wtT = jnp.transpose(Wt, (2, 0, 1))
    return _tc_last_head(p1, p2, g1, g2, degp1, degp2, b3.reshape(1, F3),
                         Watt, wtT, Vt.T, bt.reshape(1, T), Wfc,
                         bfc.reshape(1, T), Wsc, bsc.reshape(1, 1))


# speedup vs baseline: 1.1975x; 1.1975x over previous
import functools

import jax
import jax.numpy as jnp
from jax import lax
from jax.experimental import pallas as pl
from jax.experimental.pallas import tpu as pltpu
from jax.experimental.pallas import tpu_sc as plsc

N = 10000
E = 320000
D = 128
F1, F2, F3 = 64, 32, 16
T = 16
NP = 10240

_INFO = plsc.get_sparse_core_info()
NC = _INFO.num_cores
NS = _INFO.num_subcores
NW = NC * NS
EW = E // NW
C = 125
NCH = EW // C
K = 4
NT = NCH // K
RPT = NP // NS

_mesh = plsc.VectorSubcoreMesh(core_axis_name="c", subcore_axis_name="s")


@functools.partial(
    pl.kernel,
    out_type=jax.ShapeDtypeStruct((2 * NP, 16), jnp.float32),
    mesh=_mesh,
    compiler_params=pltpu.CompilerParams(use_tc_tiling_on_sc=False),
    scratch_types=[
        pltpu.VMEM((NCH, C), jnp.int32),
        pltpu.VMEM((C, 16), jnp.float32),
        pltpu.VMEM_SHARED((NP, 16), jnp.float32),
        pltpu.SemaphoreType.DMA,
    ],
)
def _deg_call(dst_hbm, ones_hbm, zrow_hbm, degp_hbm, dsta, onesv, acc, sem):
    c = lax.axis_index("c")
    s = lax.axis_index("s")
    wid = s * NC + c

    pltpu.sync_copy(zrow_hbm, acc.at[pl.ds(s * RPT, RPT)])
    pltpu.sync_copy(ones_hbm, onesv)
    pltpu.sync_copy(dst_hbm.at[pl.ds(wid * NCH, NCH)], dsta)
    plsc.subcore_barrier()

    for grp in range(0, NCH, 20):
        descs = [pltpu.async_copy(onesv, acc.at[dsta.at[j]], sem, add=True)
                 for j in range(grp, grp + 20)]
        for d in descs:
            d.wait()

    plsc.subcore_barrier()
    pltpu.sync_copy(acc.at[pl.ds(s * RPT, RPT)],
                    degp_hbm.at[pl.ds(c * NP + s * RPT, RPT)])


def _sc_scatter_body(src_hbm, dst_hbm, g_hbm, zrow_hbm, out_hbm,
                     srca, dsta, rows, acc, gsems, ssems):
    c = lax.axis_index("c")
    s = lax.axis_index("s")
    wid = s * NC + c

    pltpu.sync_copy(zrow_hbm, acc.at[pl.ds(s * RPT, RPT)])
    pltpu.sync_copy(src_hbm.at[pl.ds(wid * NCH, NCH)], srca)
    pltpu.sync_copy(dst_hbm.at[pl.ds(wid * NCH, NCH)], dsta)
    plsc.subcore_barrier()

    def fire(t):
        p = t & 1
        return [pltpu.async_copy(g_hbm.at[srca.at[t * K + b]],
                                 rows.at[p, b], gsems[p])
                for b in range(K)]

    gd = {0: fire(0), 1: fire(1)}
    for t in range(NT):
        p = t & 1
        for d in gd.pop(t):
            d.wait()
        sd = [pltpu.async_copy(rows.at[p, b],
                               acc.at[dsta.at[t * K + b]],
                               ssems[p], add=True)
              for b in range(K)]
        for d in sd:
            d.wait()
        if t + 2 < NT:
            gd[t + 2] = fire(t + 2)

    plsc.subcore_barrier()
    pltpu.sync_copy(acc.at[pl.ds(s * RPT, RPT)],
                    out_hbm.at[pl.ds(c * NP + s * RPT, RPT)])


def _make_scatter(F):
    @functools.partial(
        pl.kernel,
        out_type=jax.ShapeDtypeStruct((2 * NP, F), jnp.float32),
        mesh=_mesh,
        compiler_params=pltpu.CompilerParams(use_tc_tiling_on_sc=False),
        scratch_types=[
            pltpu.VMEM((NCH, C), jnp.int32),
            pltpu.VMEM((NCH, C), jnp.int32),
            pltpu.VMEM((2, K, C, F), jnp.float32),
            pltpu.VMEM_SHARED((NP, F), jnp.float32),
            (pltpu.SemaphoreType.DMA, pltpu.SemaphoreType.DMA),
            (pltpu.SemaphoreType.DMA, pltpu.SemaphoreType.DMA),
        ],
    )
    def call(src_hbm, dst_hbm, g_hbm, zrow_hbm, out_hbm,
             srca, dsta, rows, acc, gsems, ssems):
        _sc_scatter_body(src_hbm, dst_hbm, g_hbm, zrow_hbm, out_hbm,
                         srca, dsta, rows, acc, gsems, ssems)

    return call


_scatter = {F: _make_scatter(F) for F in (F1, F2, F3)}


def _dinv_of(degp_ref):
    deg = degp_ref[pl.ds(0, N)] + degp_ref[pl.ds(NP, N)] + 1.0
    return lax.rsqrt(jnp.maximum(deg, 1.0))


def _tc_prep_body(x_ref, w_ref, degp_ref, g_ref):
    g_ref[...] = jnp.dot(x_ref[...], w_ref[...],
                         preferred_element_type=jnp.float32) * _dinv_of(degp_ref)


def _tc_comb_body(p_ref, g_ref, degp_ref, b_ref, w_ref, o_ref):
    dinv = _dinv_of(degp_ref)
    srow = p_ref[pl.ds(0, N)] + p_ref[pl.ds(NP, N)] + g_ref[...]
    x = jnp.maximum(srow * dinv + b_ref[...], 0.0)
    o_ref[...] = jnp.dot(x, w_ref[...], preferred_element_type=jnp.float32) * dinv


def _tc_last_head_body(p1_ref, p2_ref, g1_ref, g2_ref, degp1_ref, degp2_ref,
                       b_ref, watt_ref, wtT_ref, vtT_ref, bt_ref,
                       wfc_ref, bfc_ref, wsc_ref, bsc_ref, o_ref):
    b = b_ref[...]
    watt = watt_ref[...]

    def pool(p_ref, g_ref, degp_ref):
        dinv = _dinv_of(degp_ref)
        h = (p_ref[pl.ds(0, N)] + p_ref[pl.ds(NP, N)] + g_ref[...]) * dinv + b
        mean = jnp.sum(h, axis=0, keepdims=True) * (1.0 / N)
        ctx = jnp.tanh(jnp.dot(mean, watt, preferred_element_type=jnp.float32))
        sc = jax.nn.sigmoid(jnp.sum(h * ctx, axis=1, keepdims=True))
        return jnp.sum(h * sc, axis=0, keepdims=True)

    e1 = pool(p1_ref, g1_ref, degp1_ref)
    e2 = pool(p2_ref, g2_ref, degp2_ref)
    parts = []
    for k in range(T):
        a = jnp.dot(e1, wtT_ref[k], preferred_element_type=jnp.float32)
        parts.append(jnp.sum(a * e2, axis=1, keepdims=True))
    scoring = jnp.concatenate(parts, axis=1)
    e12 = jnp.concatenate([e1, e2], axis=1)
    block = jnp.dot(e12, vtT_ref[...], preferred_element_type=jnp.float32) \
        + bt_ref[...]
    combined = jnp.maximum(scoring + block, 0.0)
    feat = jnp.maximum(
        jnp.dot(combined, wfc_ref[...], preferred_element_type=jnp.float32)
        + bfc_ref[...], 0.0)
    o_ref[...] = jax.nn.sigmoid(
        jnp.dot(feat, wsc_ref[...], preferred_element_type=jnp.float32)
        + bsc_ref[...])


_TC_PARAMS = pltpu.CompilerParams(vmem_limit_bytes=100 * 1024 * 1024)


def _tc_prep(x, w, degp):
    return pl.pallas_call(
        _tc_prep_body,
        out_shape=jax.ShapeDtypeStruct((N, w.shape[1]), jnp.float32),
        compiler_params=_TC_PARAMS,
    )(x, w, degp)


def _tc_comb(p, g, degp, b, w):
    return pl.pallas_call(
        _tc_comb_body,
        out_shape=jax.ShapeDtypeStruct((N, w.shape[1]), jnp.float32),
        compiler_params=_TC_PARAMS,
    )(p, g, degp, b, w)


def _tc_last_head(p1, p2, g1, g2, degp1, degp2, b, watt, wtT, vtT, bt,
                  wfc, bfc, wsc, bsc):
    return pl.pallas_call(
        _tc_last_head_body,
        out_shape=jax.ShapeDtypeStruct((1, 1), jnp.float32),
        compiler_params=_TC_PARAMS,
    )(p1, p2, g1, g2, degp1, degp2, b, watt, wtT, vtT, bt, wfc, bfc, wsc, bsc)


def kernel(features_1, edge_index_1, batch_1, features_2, edge_index_2,
           batch_2, W1, b1, W2, b2, W3, b3, Watt, Wt, Vt, bt, Wfc, bfc,
           Wsc, bsc):
    src1 = edge_index_1[0].reshape(E // C, C)
    dst1 = edge_index_1[1].reshape(E // C, C)
    src2 = edge_index_2[0].reshape(E // C, C)
    dst2 = edge_index_2[1].reshape(E // C, C)

    ones16 = jnp.ones((C, 16), jnp.float32)
    zrow = {F: jnp.zeros((RPT, F), jnp.float32) for F in (16, F1, F2, F3)}

    degp1 = _deg_call(dst1, ones16, zrow[16])
    degp2 = _deg_call(dst2, ones16, zrow[16])
    degc1 = degp1[:, :1]
    degc2 = degp2[:, :1]

    g1 = _tc_prep(features_1, W1, degc1)
    g2 = _tc_prep(features_2, W1, degc2)
    for b, wn, F in ((b1.reshape(1, F1), W2, F1), (b2.reshape(1, F2), W3, F2)):
        p1 = _scatter[F](src1, dst1, g1, zrow[F])
        p2 = _scatter[F](src2, dst2, g2, zrow[F])
        g1 = _tc_comb(p1, g1, degc1, b, wn)
        g2 = _tc_comb(p2, g2, degc2, b, wn)
    p1 = _scatter[F3](src1, dst1, g1, zrow[F3])
    p2 = _scatter[F3](src2, dst2, g2, zrow[F3])

    wtT = jnp.transpose(Wt, (2, 0, 1))
    return _tc_last_head(p1, p2, g1, g2, degc1, degc2, b3.reshape(1, F3),
                         Watt, wtT, Vt.T, bt.reshape(1, T), Wfc,
                         bfc.reshape(1, T), Wsc, bsc.reshape(1, 1))


# speedup vs baseline: 1.2379x; 1.0337x over previous
import functools

import jax
import jax.numpy as jnp
from jax import lax
from jax.experimental import pallas as pl
from jax.experimental.pallas import tpu as pltpu
from jax.experimental.pallas import tpu_sc as plsc

N = 10000
E = 320000
D = 128
F1, F2, F3 = 64, 32, 16
T = 16
NP = 10240

_INFO = plsc.get_sparse_core_info()
NC = _INFO.num_cores
NS = _INFO.num_subcores
EW = E // NS
C = 125
NCH = EW // C
IH = 2
NCHh = NCH // IH
K = 4
NT = NCHh // K
RPT = NP // NS
NCA = E // C

_mesh = plsc.VectorSubcoreMesh(core_axis_name="c", subcore_axis_name="s")


@functools.partial(
    pl.kernel,
    out_type=jax.ShapeDtypeStruct((2, NP, 16), jnp.float32),
    mesh=_mesh,
    compiler_params=pltpu.CompilerParams(use_tc_tiling_on_sc=False),
    scratch_types=[
        pltpu.VMEM((NCH, C), jnp.int32),
        pltpu.VMEM((C, 16), jnp.float32),
        pltpu.VMEM_SHARED((NP, 16), jnp.float32),
        pltpu.SemaphoreType.DMA,
    ],
)
def _deg_call(dst_hbm, ones_hbm, zrow_hbm, degp_hbm, dsta, onesv, acc, sem):
    c = lax.axis_index("c")
    s = lax.axis_index("s")

    pltpu.sync_copy(zrow_hbm, acc.at[pl.ds(s * RPT, RPT)])
    pltpu.sync_copy(ones_hbm, onesv)
    pltpu.sync_copy(dst_hbm.at[c, pl.ds(s * NCH, NCH)], dsta)
    plsc.subcore_barrier()

    for grp in range(0, NCH, 20):
        descs = [pltpu.async_copy(onesv, acc.at[dsta.at[j]], sem, add=True)
                 for j in range(grp, grp + 20)]
        for d in descs:
            d.wait()

    plsc.subcore_barrier()
    pltpu.sync_copy(acc.at[pl.ds(s * RPT, RPT)],
                    degp_hbm.at[c, pl.ds(s * RPT, RPT)])


def _sc_scatter_body(src_hbm, dst_hbm, g_hbm, zrow_hbm, out_hbm,
                     srca, dsta, rows, acc, gsems, ssems):
    c = lax.axis_index("c")
    s = lax.axis_index("s")

    pltpu.sync_copy(zrow_hbm, acc.at[pl.ds(s * RPT, RPT)])
    plsc.subcore_barrier()

    for h in range(IH):
        pltpu.sync_copy(src_hbm.at[c, pl.ds(s * NCH + h * NCHh, NCHh)], srca)
        pltpu.sync_copy(dst_hbm.at[c, pl.ds(s * NCH + h * NCHh, NCHh)], dsta)

        def fire(t):
            p = t & 1
            return [pltpu.async_copy(g_hbm.at[c].at[srca.at[t * K + b]],
                                     rows.at[p, b], gsems[p])
                    for b in range(K)]

        gd = {0: fire(0), 1: fire(1)}
        for t in range(NT):
            p = t & 1
            for d in gd.pop(t):
                d.wait()
            sd = [pltpu.async_copy(rows.at[p, b],
                                   acc.at[dsta.at[t * K + b]],
                                   ssems[p], add=True)
                  for b in range(K)]
            for d in sd:
                d.wait()
            if t + 2 < NT:
                gd[t + 2] = fire(t + 2)

    plsc.subcore_barrier()
    pltpu.sync_copy(acc.at[pl.ds(s * RPT, RPT)],
                    out_hbm.at[c, pl.ds(s * RPT, RPT)])


def _make_scatter(F):
    @functools.partial(
        pl.kernel,
        out_type=jax.ShapeDtypeStruct((2, NP, F), jnp.float32),
        mesh=_mesh,
        compiler_params=pltpu.CompilerParams(use_tc_tiling_on_sc=False),
        scratch_types=[
            pltpu.VMEM((NCHh, C), jnp.int32),
            pltpu.VMEM((NCHh, C), jnp.int32),
            pltpu.VMEM((2, K, C, F), jnp.float32),
            pltpu.VMEM_SHARED((NP, F), jnp.float32),
            (pltpu.SemaphoreType.DMA, pltpu.SemaphoreType.DMA),
            (pltpu.SemaphoreType.DMA, pltpu.SemaphoreType.DMA),
        ],
    )
    def call(src_hbm, dst_hbm, g_hbm, zrow_hbm, out_hbm,
             srca, dsta, rows, acc, gsems, ssems):
        _sc_scatter_body(src_hbm, dst_hbm, g_hbm, zrow_hbm, out_hbm,
                         srca, dsta, rows, acc, gsems, ssems)

    return call


_scatter = {F: _make_scatter(F) for F in (F1, F2, F3)}


def _dinv_of(degp_ref):
    deg = degp_ref[pl.ds(0, N)] + 1.0
    return lax.rsqrt(jnp.maximum(deg, 1.0))


def _tc_prep_body(x1_ref, x2_ref, w_ref, degp1_ref, degp2_ref, g_ref):
    w = w_ref[...]
    g_ref[0] = jnp.dot(x1_ref[...], w,
                       preferred_element_type=jnp.float32) * _dinv_of(degp1_ref)
    g_ref[1] = jnp.dot(x2_ref[...], w,
                       preferred_element_type=jnp.float32) * _dinv_of(degp2_ref)


def _tc_comb_body(p_ref, g_ref, degp1_ref, degp2_ref, b_ref, w_ref, o_ref):
    w = w_ref[...]
    b = b_ref[...]
    for i, degp_ref in ((0, degp1_ref), (1, degp2_ref)):
        dinv = _dinv_of(degp_ref)
        srow = p_ref[i, pl.ds(0, N)] + g_ref[i]
        x = jnp.maximum(srow * dinv + b, 0.0)
        o_ref[i] = jnp.dot(x, w, preferred_element_type=jnp.float32) * dinv


def _tc_last_head_body(p_ref, g_ref, degp1_ref, degp2_ref,
                       b_ref, watt_ref, wtT_ref, vtT_ref, bt_ref,
                       wfc_ref, bfc_ref, wsc_ref, bsc_ref, o_ref):
    b = b_ref[...]
    watt = watt_ref[...]

    def pool(i, degp_ref):
        dinv = _dinv_of(degp_ref)
        h = (p_ref[i, pl.ds(0, N)] + g_ref[i]) * dinv + b
        mean = jnp.sum(h, axis=0, keepdims=True) * (1.0 / N)
        ctx = jnp.tanh(jnp.dot(mean, watt, preferred_element_type=jnp.float32))
        sc = jax.nn.sigmoid(jnp.sum(h * ctx, axis=1, keepdims=True))
        return jnp.sum(h * sc, axis=0, keepdims=True)

    e1 = pool(0, degp1_ref)
    e2 = pool(1, degp2_ref)
    parts = []
    for k in range(T):
        a = jnp.dot(e1, wtT_ref[k], preferred_element_type=jnp.float32)
        parts.append(jnp.sum(a * e2, axis=1, keepdims=True))
    scoring = jnp.concatenate(parts, axis=1)
    e12 = jnp.concatenate([e1, e2], axis=1)
    block = jnp.dot(e12, vtT_ref[...], preferred_element_type=jnp.float32) \
        + bt_ref[...]
    combined = jnp.maximum(scoring + block, 0.0)
    feat = jnp.maximum(
        jnp.dot(combined, wfc_ref[...], preferred_element_type=jnp.float32)
        + bfc_ref[...], 0.0)
    o_ref[...] = jax.nn.sigmoid(
        jnp.dot(feat, wsc_ref[...], preferred_element_type=jnp.float32)
        + bsc_ref[...])


_TC_PARAMS = pltpu.CompilerParams(vmem_limit_bytes=100 * 1024 * 1024)


def _tc_prep(x1, x2, w, degp1, degp2):
    return pl.pallas_call(
        _tc_prep_body,
        out_shape=jax.ShapeDtypeStruct((2, N, w.shape[1]), jnp.float32),
        compiler_params=_TC_PARAMS,
    )(x1, x2, w, degp1, degp2)


def _tc_comb(p, g, degp1, degp2, b, w):
    return pl.pallas_call(
        _tc_comb_body,
        out_shape=jax.ShapeDtypeStruct((2, N, w.shape[1]), jnp.float32),
        compiler_params=_TC_PARAMS,
    )(p, g, degp1, degp2, b, w)


def _tc_last_head(p, g, degp1, degp2, b, watt, wtT, vtT, bt,
                  wfc, bfc, wsc, bsc):
    return pl.pallas_call(
        _tc_last_head_body,
        out_shape=jax.ShapeDtypeStruct((1, 1), jnp.float32),
        compiler_params=_TC_PARAMS,
    )(p, g, degp1, degp2, b, watt, wtT, vtT, bt, wfc, bfc, wsc, bsc)


def kernel(features_1, edge_index_1, batch_1, features_2, edge_index_2,
           batch_2, W1, b1, W2, b2, W3, b3, Watt, Wt, Vt, bt, Wfc, bfc,
           Wsc, bsc):
    src12 = jnp.stack([edge_index_1[0].reshape(NCA, C),
                       edge_index_2[0].reshape(NCA, C)])
    dst12 = jnp.stack([edge_index_1[1].reshape(NCA, C),
                       edge_index_2[1].reshape(NCA, C)])

    ones16 = jnp.ones((C, 16), jnp.float32)
    zrow = {F: jnp.zeros((RPT, F), jnp.float32) for F in (16, F1, F2, F3)}

    degp = _deg_call(dst12, ones16, zrow[16])
    degc1 = degp[0, :, :1]
    degc2 = degp[1, :, :1]

    g = _tc_prep(features_1, features_2, W1, degc1, degc2)
    for b, wn, F in ((b1.reshape(1, F1), W2, F1), (b2.reshape(1, F2), W3, F2)):
        p = _scatter[F](src12, dst12, g, zrow[F])
        g = _tc_comb(p, g, degc1, degc2, b, wn)
    p = _scatter[F3](src12, dst12, g, zrow[F3])

    wtT = jnp.transpose(Wt, (2, 0, 1))
    return _tc_last_head(p, g, degc1, degc2, b3.reshape(1, F3),
                         Watt, wtT, Vt.T, bt.reshape(1, T), Wfc,
                         bfc.reshape(1, T), Wsc, bsc.reshape(1, 1))


# speedup vs baseline: 1.2529x; 1.0121x over previous
import functools

import jax
import jax.numpy as jnp
from jax import lax
from jax.experimental import pallas as pl
from jax.experimental.pallas import tpu as pltpu
from jax.experimental.pallas import tpu_sc as plsc

N = 10000
E = 320000
D = 128
F1, F2, F3 = 64, 32, 16
T = 16
NP = 10240

_INFO = plsc.get_sparse_core_info()
NC = _INFO.num_cores
NS = _INFO.num_subcores
EW = E // NS
C = 125
NCH = EW // C
IH = 2
NCHh = NCH // IH
K = 4
RPT = NP // NS
NCA = E // C

_mesh = plsc.VectorSubcoreMesh(core_axis_name="c", subcore_axis_name="s")


@functools.partial(
    pl.kernel,
    out_type=jax.ShapeDtypeStruct((2, NP, 16), jnp.float32),
    mesh=_mesh,
    compiler_params=pltpu.CompilerParams(use_tc_tiling_on_sc=False),
    scratch_types=[
        pltpu.VMEM((NCH, C), jnp.int32),
        pltpu.VMEM((C, 16), jnp.float32),
        pltpu.VMEM_SHARED((NP, 16), jnp.float32),
        pltpu.SemaphoreType.DMA,
    ],
)
def _deg_call(dst_hbm, ones_hbm, zrow_hbm, degp_hbm, dsta, onesv, acc, sem):
    c = lax.axis_index("c")
    s = lax.axis_index("s")

    pltpu.sync_copy(zrow_hbm, acc.at[pl.ds(s * RPT, RPT)])
    pltpu.sync_copy(ones_hbm, onesv)
    pltpu.sync_copy(dst_hbm.at[c, pl.ds(s * NCH, NCH)], dsta)
    plsc.subcore_barrier()

    for grp in range(0, NCH, 20):
        descs = [pltpu.async_copy(onesv, acc.at[dsta.at[j]], sem, add=True)
                 for j in range(grp, grp + 20)]
        for d in descs:
            d.wait()

    plsc.subcore_barrier()
    pltpu.sync_copy(acc.at[pl.ds(s * RPT, RPT)],
                    degp_hbm.at[c, pl.ds(s * RPT, RPT)])


def _sc_scatter_body(src_hbm, dst_hbm, g_hbm, zrow_hbm, out_hbm,
                     srca, dsta, rows, acc, gsems, ssems, k):
    c = lax.axis_index("c")
    s = lax.axis_index("s")
    nt = NCHh // k

    pltpu.sync_copy(zrow_hbm, acc.at[pl.ds(s * RPT, RPT)])
    plsc.subcore_barrier()

    for h in range(IH):
        pltpu.sync_copy(src_hbm.at[c, pl.ds(s * NCH + h * NCHh, NCHh)], srca)
        pltpu.sync_copy(dst_hbm.at[c, pl.ds(s * NCH + h * NCHh, NCHh)], dsta)

        def fire(t):
            p = t & 1
            return [pltpu.async_copy(g_hbm.at[c].at[srca.at[t * k + b]],
                                     rows.at[p, b], gsems[p])
                    for b in range(k)]

        gd = {0: fire(0), 1: fire(1)}
        for t in range(nt):
            p = t & 1
            for d in gd.pop(t):
                d.wait()
            sd = [pltpu.async_copy(rows.at[p, b],
                                   acc.at[dsta.at[t * k + b]],
                                   ssems[p], add=True)
                  for b in range(k)]
            for d in sd:
                d.wait()
            if t + 2 < nt:
                gd[t + 2] = fire(t + 2)

    plsc.subcore_barrier()
    pltpu.sync_copy(acc.at[pl.ds(s * RPT, RPT)],
                    out_hbm.at[c, pl.ds(s * RPT, RPT)])


def _make_scatter(F, k=K):
    @functools.partial(
        pl.kernel,
        out_type=jax.ShapeDtypeStruct((2, NP, F), jnp.float32),
        mesh=_mesh,
        compiler_params=pltpu.CompilerParams(use_tc_tiling_on_sc=False),
        scratch_types=[
            pltpu.VMEM((NCHh, C), jnp.int32),
            pltpu.VMEM((NCHh, C), jnp.int32),
            pltpu.VMEM((2, k, C, F), jnp.float32),
            pltpu.VMEM_SHARED((NP, F), jnp.float32),
            (pltpu.SemaphoreType.DMA, pltpu.SemaphoreType.DMA),
            (pltpu.SemaphoreType.DMA, pltpu.SemaphoreType.DMA),
        ],
    )
    def call(src_hbm, dst_hbm, g_hbm, zrow_hbm, out_hbm,
             srca, dsta, rows, acc, gsems, ssems):
        _sc_scatter_body(src_hbm, dst_hbm, g_hbm, zrow_hbm, out_hbm,
                         srca, dsta, rows, acc, gsems, ssems, k)

    return call


_scatter = {F1: _make_scatter(F1, 4),
            F2: _make_scatter(F2, 8),
            F3: _make_scatter(F3, 8)}


def _dinv_of(degp_ref):
    deg = degp_ref[pl.ds(0, N)] + 1.0
    return lax.rsqrt(jnp.maximum(deg, 1.0))


def _tc_prep_body(x1_ref, x2_ref, w_ref, degp1_ref, degp2_ref, g_ref):
    w = w_ref[...]
    g_ref[0] = jnp.dot(x1_ref[...], w,
                       preferred_element_type=jnp.float32) * _dinv_of(degp1_ref)
    g_ref[1] = jnp.dot(x2_ref[...], w,
                       preferred_element_type=jnp.float32) * _dinv_of(degp2_ref)


def _tc_comb_body(p_ref, g_ref, degp1_ref, degp2_ref, b_ref, w_ref, o_ref):
    w = w_ref[...]
    b = b_ref[...]
    for i, degp_ref in ((0, degp1_ref), (1, degp2_ref)):
        dinv = _dinv_of(degp_ref)
        srow = p_ref[i, pl.ds(0, N)] + g_ref[i]
        x = jnp.maximum(srow * dinv + b, 0.0)
        o_ref[i] = jnp.dot(x, w, preferred_element_type=jnp.float32) * dinv


def _tc_last_head_body(p_ref, g_ref, degp1_ref, degp2_ref,
                       b_ref, watt_ref, wtT_ref, vtT_ref, bt_ref,
                       wfc_ref, bfc_ref, wsc_ref, bsc_ref, o_ref):
    b = b_ref[...]
    watt = watt_ref[...]

    def pool(i, degp_ref):
        dinv = _dinv_of(degp_ref)
        h = (p_ref[i, pl.ds(0, N)] + g_ref[i]) * dinv + b
        mean = jnp.sum(h, axis=0, keepdims=True) * (1.0 / N)
        ctx = jnp.tanh(jnp.dot(mean, watt, preferred_element_type=jnp.float32))
        sc = jax.nn.sigmoid(jnp.sum(h * ctx, axis=1, keepdims=True))
        return jnp.sum(h * sc, axis=0, keepdims=True)

    e1 = pool(0, degp1_ref)
    e2 = pool(1, degp2_ref)
    parts = []
    for k in range(T):
        a = jnp.dot(e1, wtT_ref[k], preferred_element_type=jnp.float32)
        parts.append(jnp.sum(a * e2, axis=1, keepdims=True))
    scoring = jnp.concatenate(parts, axis=1)
    e12 = jnp.concatenate([e1, e2], axis=1)
    block = jnp.dot(e12, vtT_ref[...], preferred_element_type=jnp.float32) \
        + bt_ref[...]
    combined = jnp.maximum(scoring + block, 0.0)
    feat = jnp.maximum(
        jnp.dot(combined, wfc_ref[...], preferred_element_type=jnp.float32)
        + bfc_ref[...], 0.0)
    o_ref[...] = jax.nn.sigmoid(
        jnp.dot(feat, wsc_ref[...], preferred_element_type=jnp.float32)
        + bsc_ref[...])


_TC_PARAMS = pltpu.CompilerParams(vmem_limit_bytes=100 * 1024 * 1024)


def _tc_prep(x1, x2, w, degp1, degp2):
    return pl.pallas_call(
        _tc_prep_body,
        out_shape=jax.ShapeDtypeStruct((2, N, w.shape[1]), jnp.float32),
        compiler_params=_TC_PARAMS,
    )(x1, x2, w, degp1, degp2)


def _tc_comb(p, g, degp1, degp2, b, w):
    return pl.pallas_call(
        _tc_comb_body,
        out_shape=jax.ShapeDtypeStruct((2, N, w.shape[1]), jnp.float32),
        compiler_params=_TC_PARAMS,
    )(p, g, degp1, degp2, b, w)


def _tc_last_head(p, g, degp1, degp2, b, watt, wtT, vtT, bt,
                  wfc, bfc, wsc, bsc):
    return pl.pallas_call(
        _tc_last_head_body,
        out_shape=jax.ShapeDtypeStruct((1, 1), jnp.float32),
        compiler_params=_TC_PARAMS,
    )(p, g, degp1, degp2, b, watt, wtT, vtT, bt, wfc, bfc, wsc, bsc)


def kernel(features_1, edge_index_1, batch_1, features_2, edge_index_2,
           batch_2, W1, b1, W2, b2, W3, b3, Watt, Wt, Vt, bt, Wfc, bfc,
           Wsc, bsc):
    src12 = jnp.stack([edge_index_1[0].reshape(NCA, C),
                       edge_index_2[0].reshape(NCA, C)])
    dst12 = jnp.stack([edge_index_1[1].reshape(NCA, C),
                       edge_index_2[1].reshape(NCA, C)])

    ones16 = jnp.ones((C, 16), jnp.float32)
    zrow = {F: jnp.zeros((RPT, F), jnp.float32) for F in (16, F1, F2, F3)}

    degp = _deg_call(dst12, ones16, zrow[16])
    degc1 = degp[0, :, :1]
    degc2 = degp[1, :, :1]

    g = _tc_prep(features_1, features_2, W1, degc1, degc2)
    for b, wn, F in ((b1.reshape(1, F1), W2, F1), (b2.reshape(1, F2), W3, F2)):
        p = _scatter[F](src12, dst12, g, zrow[F])
        g = _tc_comb(p, g, degc1, degc2, b, wn)
    p = _scatter[F3](src12, dst12, g, zrow[F3])

    wtT = jnp.transpose(Wt, (2, 0, 1))
    return _tc_last_head(p, g, degc1, degc2, b3.reshape(1, F3),
                         Watt, wtT, Vt.T, bt.reshape(1, T), Wfc,
                         bfc.reshape(1, T), Wsc, bsc.reshape(1, 1))


# speedup vs baseline: 1.2714x; 1.0147x over previous
import functools

import jax
import jax.numpy as jnp
from jax import lax
from jax.experimental import pallas as pl
from jax.experimental.pallas import tpu as pltpu
from jax.experimental.pallas import tpu_sc as plsc

N = 10000
E = 320000
D = 128
F1, F2, F3 = 64, 32, 16
T = 16
NP = 10240

_INFO = plsc.get_sparse_core_info()
NC = _INFO.num_cores
NS = _INFO.num_subcores
EW = E // NS
C = 125
NCH = EW // C
IH = 2
NCHh = NCH // IH
K = 4
RPT = NP // NS
NCA = E // C

_mesh = plsc.VectorSubcoreMesh(core_axis_name="c", subcore_axis_name="s")


@functools.partial(
    pl.kernel,
    out_type=jax.ShapeDtypeStruct((2, NP, 8), jnp.float32),
    mesh=_mesh,
    compiler_params=pltpu.CompilerParams(use_tc_tiling_on_sc=False),
    scratch_types=[
        pltpu.VMEM((NCH, C), jnp.int32),
        pltpu.VMEM((C, 8), jnp.float32),
        pltpu.VMEM_SHARED((NP, 8), jnp.float32),
        pltpu.SemaphoreType.DMA,
    ],
)
def _deg_call(dst_hbm, ones_hbm, zrow_hbm, degp_hbm, dsta, onesv, acc, sem):
    c = lax.axis_index("c")
    s = lax.axis_index("s")

    pltpu.sync_copy(zrow_hbm, acc.at[pl.ds(s * RPT, RPT)])
    pltpu.sync_copy(ones_hbm, onesv)
    pltpu.sync_copy(dst_hbm.at[c, pl.ds(s * NCH, NCH)], dsta)
    plsc.subcore_barrier()

    for grp in range(0, NCH, 20):
        descs = [pltpu.async_copy(onesv, acc.at[dsta.at[j]], sem, add=True)
                 for j in range(grp, grp + 20)]
        for d in descs:
            d.wait()

    plsc.subcore_barrier()
    pltpu.sync_copy(acc.at[pl.ds(s * RPT, RPT)],
                    degp_hbm.at[c, pl.ds(s * RPT, RPT)])


def _sc_scatter_body(src_hbm, dst_hbm, g_hbm, zrow_hbm, out_hbm,
                     srca, dsta, rows, acc, gsems, ssems, k):
    c = lax.axis_index("c")
    s = lax.axis_index("s")
    nt = NCHh // k

    pltpu.sync_copy(zrow_hbm, acc.at[pl.ds(s * RPT, RPT)])
    plsc.subcore_barrier()

    for h in range(IH):
        pltpu.sync_copy(src_hbm.at[c, pl.ds(s * NCH + h * NCHh, NCHh)], srca)
        pltpu.sync_copy(dst_hbm.at[c, pl.ds(s * NCH + h * NCHh, NCHh)], dsta)

        def fire(t):
            p = t & 1
            return [pltpu.async_copy(g_hbm.at[c].at[srca.at[t * k + b]],
                                     rows.at[p, b], gsems[p])
                    for b in range(k)]

        gd = {0: fire(0), 1: fire(1)}
        for t in range(nt):
            p = t & 1
            for d in gd.pop(t):
                d.wait()
            sd = [pltpu.async_copy(rows.at[p, b],
                                   acc.at[dsta.at[t * k + b]],
                                   ssems[p], add=True)
                  for b in range(k)]
            for d in sd:
                d.wait()
            if t + 2 < nt:
                gd[t + 2] = fire(t + 2)

    plsc.subcore_barrier()
    pltpu.sync_copy(acc.at[pl.ds(s * RPT, RPT)],
                    out_hbm.at[c, pl.ds(s * RPT, RPT)])


def _make_scatter(F, k=K):
    @functools.partial(
        pl.kernel,
        out_type=jax.ShapeDtypeStruct((2, NP, F), jnp.float32),
        mesh=_mesh,
        compiler_params=pltpu.CompilerParams(use_tc_tiling_on_sc=False),
        scratch_types=[
            pltpu.VMEM((NCHh, C), jnp.int32),
            pltpu.VMEM((NCHh, C), jnp.int32),
            pltpu.VMEM((2, k, C, F), jnp.float32),
            pltpu.VMEM_SHARED((NP, F), jnp.float32),
            (pltpu.SemaphoreType.DMA, pltpu.SemaphoreType.DMA),
            (pltpu.SemaphoreType.DMA, pltpu.SemaphoreType.DMA),
        ],
    )
    def call(src_hbm, dst_hbm, g_hbm, zrow_hbm, out_hbm,
             srca, dsta, rows, acc, gsems, ssems):
        _sc_scatter_body(src_hbm, dst_hbm, g_hbm, zrow_hbm, out_hbm,
                         srca, dsta, rows, acc, gsems, ssems, k)

    return call


_scatter = {F1: _make_scatter(F1, 4),
            F2: _make_scatter(F2, 8),
            F3: _make_scatter(F3, 8)}


def _dinv_of(degp_ref):
    deg = degp_ref[pl.ds(0, N)] + 1.0
    return lax.rsqrt(jnp.maximum(deg, 1.0))


def _tc_prep_body(x1_ref, x2_ref, w_ref, degp1_ref, degp2_ref, g_ref):
    w = w_ref[...]
    g_ref[0] = jnp.dot(x1_ref[...], w,
                       preferred_element_type=jnp.float32) * _dinv_of(degp1_ref)
    g_ref[1] = jnp.dot(x2_ref[...], w,
                       preferred_element_type=jnp.float32) * _dinv_of(degp2_ref)


def _tc_comb_body(p_ref, g_ref, degp1_ref, degp2_ref, b_ref, w_ref, o_ref):
    w = w_ref[...]
    b = b_ref[...]
    for i, degp_ref in ((0, degp1_ref), (1, degp2_ref)):
        dinv = _dinv_of(degp_ref)
        srow = p_ref[i, pl.ds(0, N)] + g_ref[i]
        x = jnp.maximum(srow * dinv + b, 0.0)
        o_ref[i] = jnp.dot(x, w, preferred_element_type=jnp.float32) * dinv


def _tc_last_head_body(p_ref, g_ref, degp1_ref, degp2_ref,
                       b_ref, watt_ref, wtT_ref, vtT_ref, bt_ref,
                       wfc_ref, bfc_ref, wsc_ref, bsc_ref, o_ref):
    b = b_ref[...]
    watt = watt_ref[...]

    def pool(i, degp_ref):
        dinv = _dinv_of(degp_ref)
        h = (p_ref[i, pl.ds(0, N)] + g_ref[i]) * dinv + b
        mean = jnp.sum(h, axis=0, keepdims=True) * (1.0 / N)
        ctx = jnp.tanh(jnp.dot(mean, watt, preferred_element_type=jnp.float32))
        sc = jax.nn.sigmoid(jnp.sum(h * ctx, axis=1, keepdims=True))
        return jnp.sum(h * sc, axis=0, keepdims=True)

    e1 = pool(0, degp1_ref)
    e2 = pool(1, degp2_ref)
    parts = []
    for k in range(T):
        a = jnp.dot(e1, wtT_ref[k], preferred_element_type=jnp.float32)
        parts.append(jnp.sum(a * e2, axis=1, keepdims=True))
    scoring = jnp.concatenate(parts, axis=1)
    e12 = jnp.concatenate([e1, e2], axis=1)
    block = jnp.dot(e12, vtT_ref[...], preferred_element_type=jnp.float32) \
        + bt_ref[...]
    combined = jnp.maximum(scoring + block, 0.0)
    feat = jnp.maximum(
        jnp.dot(combined, wfc_ref[...], preferred_element_type=jnp.float32)
        + bfc_ref[...], 0.0)
    o_ref[...] = jax.nn.sigmoid(
        jnp.dot(feat, wsc_ref[...], preferred_element_type=jnp.float32)
        + bsc_ref[...])


_TC_PARAMS = pltpu.CompilerParams(vmem_limit_bytes=100 * 1024 * 1024)


def _tc_prep(x1, x2, w, degp1, degp2):
    return pl.pallas_call(
        _tc_prep_body,
        out_shape=jax.ShapeDtypeStruct((2, N, w.shape[1]), jnp.float32),
        compiler_params=_TC_PARAMS,
    )(x1, x2, w, degp1, degp2)


def _tc_comb(p, g, degp1, degp2, b, w):
    return pl.pallas_call(
        _tc_comb_body,
        out_shape=jax.ShapeDtypeStruct((2, N, w.shape[1]), jnp.float32),
        compiler_params=_TC_PARAMS,
    )(p, g, degp1, degp2, b, w)


def _tc_last_head(p, g, degp1, degp2, b, watt, wtT, vtT, bt,
                  wfc, bfc, wsc, bsc):
    return pl.pallas_call(
        _tc_last_head_body,
        out_shape=jax.ShapeDtypeStruct((1, 1), jnp.float32),
        compiler_params=_TC_PARAMS,
    )(p, g, degp1, degp2, b, watt, wtT, vtT, bt, wfc, bfc, wsc, bsc)


def kernel(features_1, edge_index_1, batch_1, features_2, edge_index_2,
           batch_2, W1, b1, W2, b2, W3, b3, Watt, Wt, Vt, bt, Wfc, bfc,
           Wsc, bsc):
    src12 = jnp.stack([edge_index_1[0].reshape(NCA, C),
                       edge_index_2[0].reshape(NCA, C)])
    dst12 = jnp.stack([edge_index_1[1].reshape(NCA, C),
                       edge_index_2[1].reshape(NCA, C)])

    ones8 = jnp.ones((C, 8), jnp.float32)
    zrow = {F: jnp.zeros((RPT, F), jnp.float32) for F in (8, F1, F2, F3)}

    degp = _deg_call(dst12, ones8, zrow[8])
    degc1 = degp[0, :, :1]
    degc2 = degp[1, :, :1]

    g = _tc_prep(features_1, features_2, W1, degc1, degc2)
    for b, wn, F in ((b1.reshape(1, F1), W2, F1), (b2.reshape(1, F2), W3, F2)):
        p = _scatter[F](src12, dst12, g, zrow[F])
        g = _tc_comb(p, g, degc1, degc2, b, wn)
    p = _scatter[F3](src12, dst12, g, zrow[F3])

    wtT = jnp.transpose(Wt, (2, 0, 1))
    return _tc_last_head(p, g, degc1, degc2, b3.reshape(1, F3),
                         Watt, wtT, Vt.T, bt.reshape(1, T), Wfc,
                         bfc.reshape(1, T), Wsc, bsc.reshape(1, 1))
